# baseline (speedup 1.0000x reference)
import functools

import jax
import jax.numpy as jnp
from jax import lax
from jax.experimental import pallas as pl
from jax.experimental.pallas import tpu as pltpu
from jax.experimental.pallas import tpu_sc as plsc

_INFO = plsc.get_sparse_core_info()
_NC, _NS = _INFO.num_cores, _INFO.num_subcores
_NW = _NC * _NS
_L = 16


def _sc_gather(table, idx):
    V, D = table.shape
    B = idx.shape[0]
    b_per_w = B // _NW

    @functools.partial(
        pl.kernel,
        mesh=plsc.VectorSubcoreMesh(core_axis_name="c", subcore_axis_name="s"),
        out_type=jax.ShapeDtypeStruct((B, D), jnp.float32),
        scratch_types=[
            pltpu.VMEM((b_per_w + _L,), jnp.int32),
            pltpu.VMEM((b_per_w, D), jnp.float32),
            pltpu.SemaphoreType.DMA,
        ],
        compiler_params=pltpu.CompilerParams(use_tc_tiling_on_sc=True),
    )
    def k(table_hbm, idx_hbm, out_hbm, idx_v, out_v, sem):
        wid = lax.axis_index("s") * _NC + lax.axis_index("c")
        base = wid * b_per_w
        pltpu.sync_copy(
            idx_hbm.at[pl.ds(base, b_per_w)], idx_v.at[pl.ds(0, b_per_w)]
        )

        def body(i, _):
            q = idx_v[pl.ds(i, _L)][0]
            pltpu.async_copy(table_hbm.at[q], out_v.at[i], sem)
            return 0

        lax.fori_loop(0, b_per_w, body, 0)
        pltpu.make_async_copy(
            out_hbm.at[pl.ds(base, b_per_w)], out_v, sem
        ).wait()
        pltpu.sync_copy(out_v, out_hbm.at[pl.ds(base, b_per_w)])

    return k(table, idx)


def _pt_body(xt_ref, w_ref, b_ref, o_ref):
    o_ref[...] = (
        lax.dot_general(
            xt_ref[...],
            w_ref[...],
            (((0,), (1,)), ((), ())),
            preferred_element_type=jnp.float32,
        )
        + b_ref[...]
    )


def kernel(self_words, embed, W_in, b_in):
    B = self_words.shape[0]
    V, D = embed.shape
    idx = self_words.astype(jnp.int32)

    BK = 2048
    pt = pl.pallas_call(
        _pt_body,
        grid=(pl.cdiv(V, BK),),
        in_specs=[
            pl.BlockSpec((D, BK), lambda j: (0, j)),
            pl.BlockSpec((D, D), lambda j: (0, 0)),
            pl.BlockSpec((1, D), lambda j: (0, 0)),
        ],
        out_specs=pl.BlockSpec((BK, D), lambda j: (j, 0)),
        out_shape=jax.ShapeDtypeStruct((V, D), jnp.float32),
    )(embed.T, W_in, b_in.reshape(1, D))

    return _sc_gather(pt, idx)


# speedup vs baseline: 1.7071x; 1.7071x over previous
import functools

import jax
import jax.numpy as jnp
from jax import lax
from jax.experimental import pallas as pl
from jax.experimental.pallas import tpu as pltpu
from jax.experimental.pallas import tpu_sc as plsc

_INFO = plsc.get_sparse_core_info()
_NC, _NS = _INFO.num_cores, _INFO.num_subcores
_NW = _NC * _NS
_L = 16


def _sc_gather(table, idx):
    V, D = table.shape
    B = idx.shape[0]
    b_per_w = B // _NW

    @functools.partial(
        pl.kernel,
        mesh=plsc.VectorSubcoreMesh(core_axis_name="c", subcore_axis_name="s"),
        out_type=jax.ShapeDtypeStruct((B, D), jnp.float32),
        scratch_types=[
            pltpu.VMEM((b_per_w + _L,), jnp.int32),
            pltpu.VMEM((b_per_w, D), jnp.float32),
            pltpu.SemaphoreType.DMA,
        ],
        compiler_params=pltpu.CompilerParams(use_tc_tiling_on_sc=True),
    )
    def k(table_hbm, idx_hbm, out_hbm, idx_v, out_v, sem):
        wid = lax.axis_index("s") * _NC + lax.axis_index("c")
        base = wid * b_per_w
        pltpu.sync_copy(
            idx_hbm.at[pl.ds(base, b_per_w)], idx_v.at[pl.ds(0, b_per_w)]
        )

        def body(i, _):
            q = idx_v[pl.ds(i, _L)][0]
            pltpu.async_copy(table_hbm.at[q], out_v.at[i], sem)
            return 0

        lax.fori_loop(0, b_per_w, body, 0)
        pltpu.make_async_copy(
            out_hbm.at[pl.ds(base, b_per_w)], out_v, sem
        ).wait()
        pltpu.sync_copy(out_v, out_hbm.at[pl.ds(base, b_per_w)])

    return k(table, idx)


def _pt_body(xt_ref, w_ref, b_ref, o_ref):
    o_ref[...] = (
        lax.dot_general(
            xt_ref[...],
            w_ref[...],
            (((0,), (1,)), ((), ())),
            preferred_element_type=jnp.float32,
        )
        + b_ref[...]
    )


def kernel(self_words, embed, W_in, b_in):
    B = self_words.shape[0]
    V, D = embed.shape
    idx = self_words.astype(jnp.int32)

    BK = 8192
    pt = pl.pallas_call(
        _pt_body,
        grid=(pl.cdiv(V, BK),),
        in_specs=[
            pl.BlockSpec((D, BK), lambda j: (0, j)),
            pl.BlockSpec((D, D), lambda j: (0, 0)),
            pl.BlockSpec((1, D), lambda j: (0, 0)),
        ],
        out_specs=pl.BlockSpec((BK, D), lambda j: (j, 0)),
        out_shape=jax.ShapeDtypeStruct((V, D), jnp.float32),
        compiler_params=pltpu.CompilerParams(
            fuse_transposed_lhs_in_matmul=True,
        ),
    )(embed.T, W_in, b_in.reshape(1, D))

    return _sc_gather(pt, idx)


# speedup vs baseline: 1.8648x; 1.0923x over previous
import functools

import jax
import jax.numpy as jnp
from jax import lax
from jax.experimental import pallas as pl
from jax.experimental.pallas import tpu as pltpu
from jax.experimental.pallas import tpu_sc as plsc

_INFO = plsc.get_sparse_core_info()
_NC, _NS = _INFO.num_cores, _INFO.num_subcores
_NW = _NC * _NS
_L = 16


def _sc_gather(table, idx):
    V, D = table.shape
    B = idx.shape[0]
    b_per_w = B // _NW

    @functools.partial(
        pl.kernel,
        mesh=plsc.VectorSubcoreMesh(core_axis_name="c", subcore_axis_name="s"),
        out_type=jax.ShapeDtypeStruct((B, D), jnp.float32),
        scratch_types=[
            pltpu.VMEM((b_per_w + _L,), jnp.int32),
            pltpu.VMEM((b_per_w, D), jnp.float32),
            pltpu.SemaphoreType.DMA,
        ],
        compiler_params=pltpu.CompilerParams(use_tc_tiling_on_sc=True),
    )
    def k(table_hbm, idx_hbm, out_hbm, idx_v, out_v, sem):
        wid = lax.axis_index("s") * _NC + lax.axis_index("c")
        base = wid * b_per_w
        pltpu.sync_copy(
            idx_hbm.at[pl.ds(base, b_per_w)], idx_v.at[pl.ds(0, b_per_w)]
        )

        def body(i, _):
            q = idx_v[pl.ds(i, _L)][0]
            pltpu.async_copy(table_hbm.at[q], out_v.at[i], sem)
            return 0

        lax.fori_loop(0, b_per_w, body, 0)
        pltpu.make_async_copy(
            out_hbm.at[pl.ds(base, b_per_w)], out_v, sem
        ).wait()
        pltpu.sync_copy(out_v, out_hbm.at[pl.ds(base, b_per_w)])

    return k(table, idx)


def _pt_body(xt_ref, w_ref, b_ref, o_ref):
    o_ref[...] = (
        lax.dot_general(
            xt_ref[...],
            w_ref[...],
            (((0,), (1,)), ((), ())),
            preferred_element_type=jnp.float32,
        )
        + b_ref[...]
    )


def kernel(self_words, embed, W_in, b_in):
    B = self_words.shape[0]
    V, D = embed.shape
    idx = self_words.astype(jnp.int32)

    BK = 16384
    pt = pl.pallas_call(
        _pt_body,
        grid=(pl.cdiv(V, BK),),
        in_specs=[
            pl.BlockSpec((D, BK), lambda j: (0, j)),
            pl.BlockSpec((D, D), lambda j: (0, 0)),
            pl.BlockSpec((1, D), lambda j: (0, 0)),
        ],
        out_specs=pl.BlockSpec((BK, D), lambda j: (j, 0)),
        out_shape=jax.ShapeDtypeStruct((V, D), jnp.float32),
        compiler_params=pltpu.CompilerParams(
            fuse_transposed_lhs_in_matmul=True,
        ),
    )(embed.T, W_in, b_in.reshape(1, D))

    return _sc_gather(pt, idx)


# speedup vs baseline: 1.9027x; 1.0203x over previous
import functools

import jax
import jax.numpy as jnp
from jax import lax
from jax.experimental import pallas as pl
from jax.experimental.pallas import tpu as pltpu
from jax.experimental.pallas import tpu_sc as plsc

_INFO = plsc.get_sparse_core_info()
_NC, _NS = _INFO.num_cores, _INFO.num_subcores
_NW = _NC * _NS
_L = 16


def _sc_gather(table, idx):
    V, D = table.shape
    B = idx.shape[0]
    b_per_w = B // _NW

    @functools.partial(
        pl.kernel,
        mesh=plsc.VectorSubcoreMesh(core_axis_name="c", subcore_axis_name="s"),
        out_type=jax.ShapeDtypeStruct((B, D), jnp.float32),
        scratch_types=[
            pltpu.VMEM((b_per_w + _L,), jnp.int32),
            pltpu.VMEM((b_per_w, D), jnp.float32),
            pltpu.SemaphoreType.DMA,
        ],
        compiler_params=pltpu.CompilerParams(use_tc_tiling_on_sc=True),
    )
    def k(table_hbm, idx_hbm, out_hbm, idx_v, out_v, sem):
        wid = lax.axis_index("s") * _NC + lax.axis_index("c")
        base = wid * b_per_w
        pltpu.sync_copy(
            idx_hbm.at[pl.ds(base, b_per_w)], idx_v.at[pl.ds(0, b_per_w)]
        )

        def body(i, _):
            q = idx_v[pl.ds(i, _L)][0]
            pltpu.async_copy(table_hbm.at[q], out_v.at[i], sem)
            return 0

        lax.fori_loop(0, b_per_w, body, 0)
        pltpu.make_async_copy(
            out_hbm.at[pl.ds(base, b_per_w)], out_v, sem
        ).wait()
        pltpu.sync_copy(out_v, out_hbm.at[pl.ds(base, b_per_w)])

    return k(table, idx)


def _pt_body(xt_ref, w_ref, b_ref, o_ref):
    o_ref[...] = (
        lax.dot_general(
            xt_ref[...],
            w_ref[...],
            (((0,), (1,)), ((), ())),
            preferred_element_type=jnp.float32,
        )
        + b_ref[...]
    )


def kernel(self_words, embed, W_in, b_in):
    B = self_words.shape[0]
    V, D = embed.shape
    idx = self_words.astype(jnp.int32)

    BK = 32768
    pt = pl.pallas_call(
        _pt_body,
        grid=(pl.cdiv(V, BK),),
        in_specs=[
            pl.BlockSpec((D, BK), lambda j: (0, j)),
            pl.BlockSpec((D, D), lambda j: (0, 0)),
            pl.BlockSpec((1, D), lambda j: (0, 0)),
        ],
        out_specs=pl.BlockSpec((BK, D), lambda j: (j, 0)),
        out_shape=jax.ShapeDtypeStruct((V, D), jnp.float32),
        compiler_params=pltpu.CompilerParams(
            fuse_transposed_lhs_in_matmul=True,
            vmem_limit_bytes=100 * 1024 * 1024,
        ),
    )(embed.T, W_in, b_in.reshape(1, D))

    return _sc_gather(pt, idx)


# speedup vs baseline: 2.0134x; 1.0582x over previous
import functools

import jax
import jax.numpy as jnp
from jax import lax
from jax.experimental import pallas as pl
from jax.experimental.pallas import tpu as pltpu
from jax.experimental.pallas import tpu_sc as plsc

_INFO = plsc.get_sparse_core_info()
_NC, _NS = _INFO.num_cores, _INFO.num_subcores
_NW = _NC * _NS
_L = 16

_BK = 32768
_BKH = _BK // 2
_BK_SHIFT = 15
_BKH_SHIFT = 14
_CH = 128


def _sc_gather_fold(pt2, idx, D):
    V2, D2 = pt2.shape
    B = idx.shape[0]
    b_per_w = B // _NW

    @functools.partial(
        pl.kernel,
        mesh=plsc.VectorSubcoreMesh(core_axis_name="c", subcore_axis_name="s"),
        out_type=jax.ShapeDtypeStruct((B, D), jnp.float32),
        scratch_types=[
            pltpu.VMEM((b_per_w + _L,), jnp.int32),
            pltpu.VMEM((2, _CH, D2), jnp.float32),
            pltpu.VMEM((b_per_w, D), jnp.float32),
            pltpu.SemaphoreType.DMA,
            pltpu.SemaphoreType.DMA,
        ],
        compiler_params=pltpu.CompilerParams(use_tc_tiling_on_sc=True),
    )
    def k(pt2_hbm, idx_hbm, out_hbm, idx_v, rows_v, out_v, sem0, sem1):
        sems = (sem0, sem1)
        wid = lax.axis_index("s") * _NC + lax.axis_index("c")
        base = wid * b_per_w
        n_chunks = b_per_w // _CH
        pltpu.sync_copy(
            idx_hbm.at[pl.ds(base, b_per_w)], idx_v.at[pl.ds(0, b_per_w)]
        )

        def fire(c):
            def body(i, _, c=c):
                q = idx_v[pl.ds(c * _CH + i, _L)][0]
                j = lax.shift_right_logical(q, _BK_SHIFT)
                r = q & (_BK - 1)
                row = (j << _BKH_SHIFT) | (r & (_BKH - 1))
                pltpu.async_copy(
                    pt2_hbm.at[row], rows_v.at[c % 2].at[i], sems[c % 2]
                )
                return 0

            lax.fori_loop(0, _CH, body, 0)

        fire(0)
        for c in range(n_chunks):
            if c + 1 < n_chunks:
                fire(c + 1)
            pltpu.make_async_copy(
                pt2_hbm.at[pl.ds(0, _CH)], rows_v.at[c % 2], sems[c % 2]
            ).wait()

            def select(i, _, c=c):
                q = idx_v[pl.ds(c * _CH + i, _L)][0]
                h = lax.shift_right_logical(q, _BKH_SHIFT) & 1
                off = h * D
                for kk in range(D // _L):
                    out_v[c * _CH + i, pl.ds(kk * _L, _L)] = (
                        rows_v[c % 2, i, pl.ds(off + kk * _L, _L)]
                    )
                return 0

            lax.fori_loop(0, _CH, select, 0)
        pltpu.sync_copy(out_v, out_hbm.at[pl.ds(base, b_per_w)])

    return k(pt2, idx)


def _pt_body(xt_ref, w_ref, b_ref, o_ref):
    res = (
        lax.dot_general(
            xt_ref[...],
            w_ref[...],
            (((0,), (1,)), ((), ())),
            preferred_element_type=jnp.float32,
        )
        + b_ref[...]
    )
    d = res.shape[1]
    o_ref[:, 0:d] = res[0:_BKH, :]
    o_ref[:, d : 2 * d] = res[_BKH:, :]


def kernel(self_words, embed, W_in, b_in):
    B = self_words.shape[0]
    V, D = embed.shape
    idx = self_words.astype(jnp.int32)

    nb = pl.cdiv(V, _BK)
    pt2 = pl.pallas_call(
        _pt_body,
        grid=(nb,),
        in_specs=[
            pl.BlockSpec((D, _BK), lambda j: (0, j)),
            pl.BlockSpec((D, D), lambda j: (0, 0)),
            pl.BlockSpec((1, D), lambda j: (0, 0)),
        ],
        out_specs=pl.BlockSpec((_BKH, 2 * D), lambda j: (j, 0)),
        out_shape=jax.ShapeDtypeStruct((nb * _BKH, 2 * D), jnp.float32),
        compiler_params=pltpu.CompilerParams(
            fuse_transposed_lhs_in_matmul=True,
            vmem_limit_bytes=100 * 1024 * 1024,
        ),
    )(embed.T, W_in, b_in.reshape(1, D))

    return _sc_gather_fold(pt2, idx, D)


# speedup vs baseline: 2.3569x; 1.1706x over previous
import functools

import jax
import jax.numpy as jnp
from jax import lax
from jax.experimental import pallas as pl
from jax.experimental.pallas import tpu as pltpu
from jax.experimental.pallas import tpu_sc as plsc

_INFO = plsc.get_sparse_core_info()
_NC, _NS = _INFO.num_cores, _INFO.num_subcores
_NW = _NC * _NS
_L = 16

_BK = 32768
_BKH = _BK // 2
_BK_SHIFT = 15
_BKH_SHIFT = 14
_CH = 128


def _sc_gather_fold(pt2, idx, D):
    V2, D2 = pt2.shape
    B = idx.shape[0]
    b_per_w = B // _NW

    @functools.partial(
        pl.kernel,
        mesh=plsc.VectorSubcoreMesh(core_axis_name="c", subcore_axis_name="s"),
        out_type=jax.ShapeDtypeStruct((B, D), jnp.float32),
        scratch_types=[
            pltpu.VMEM((b_per_w + _L,), jnp.int32),
            pltpu.VMEM((2, _CH, D2), jnp.float32),
            pltpu.VMEM((b_per_w, D), jnp.float32),
            pltpu.SemaphoreType.DMA,
            pltpu.SemaphoreType.DMA,
        ],
        compiler_params=pltpu.CompilerParams(use_tc_tiling_on_sc=True),
    )
    def k(pt2_hbm, idx_hbm, out_hbm, idx_v, rows_v, out_v, sem0, sem1):
        sems = (sem0, sem1)
        wid = lax.axis_index("s") * _NC + lax.axis_index("c")
        base = wid * b_per_w
        n_chunks = b_per_w // _CH
        pltpu.sync_copy(
            idx_hbm.at[pl.ds(base, b_per_w)], idx_v.at[pl.ds(0, b_per_w)]
        )

        def fire(c):
            def body(i, _, c=c):
                q = idx_v[pl.ds(c * _CH + i, _L)][0]
                j = lax.shift_right_logical(q, _BK_SHIFT)
                r = q & (_BK - 1)
                row = (j << _BKH_SHIFT) | (r & (_BKH - 1))
                pltpu.async_copy(
                    pt2_hbm.at[row], rows_v.at[c % 2].at[i], sems[c % 2]
                )
                return 0

            lax.fori_loop(0, _CH, body, 0)

        fire(0)
        for c in range(n_chunks):
            if c + 1 < n_chunks:
                fire(c + 1)
            pltpu.make_async_copy(
                pt2_hbm.at[pl.ds(0, _CH)], rows_v.at[c % 2], sems[c % 2]
            ).wait()

            def select(i, _, c=c):
                q = idx_v[pl.ds(c * _CH + i, _L)][0]
                h = lax.shift_right_logical(q, _BKH_SHIFT) & 1
                off = h * D
                for kk in range(D // _L):
                    out_v[c * _CH + i, pl.ds(kk * _L, _L)] = (
                        rows_v[c % 2, i, pl.ds(off + kk * _L, _L)]
                    )
                return 0

            lax.fori_loop(0, _CH, select, 0)
        pltpu.sync_copy(out_v, out_hbm.at[pl.ds(base, b_per_w)])

    return k(pt2, idx)


def _pt_body(xt_ref, w_ref, b_ref, o_ref):
    d = w_ref.shape[0]
    wt = w_ref[...].T.astype(jnp.bfloat16)
    for h in range(2):
        o_ref[:, h * d : (h + 1) * d] = (
            lax.dot_general(
                xt_ref[:, h * _BKH : (h + 1) * _BKH].astype(jnp.bfloat16),
                wt,
                (((0,), (0,)), ((), ())),
                preferred_element_type=jnp.float32,
            )
            + b_ref[...]
        )


def kernel(self_words, embed, W_in, b_in):
    B = self_words.shape[0]
    V, D = embed.shape
    idx = self_words.astype(jnp.int32)

    nb = pl.cdiv(V, _BK)
    pt2 = pl.pallas_call(
        _pt_body,
        grid=(nb,),
        in_specs=[
            pl.BlockSpec((D, _BK), lambda j: (0, j)),
            pl.BlockSpec((D, D), lambda j: (0, 0)),
            pl.BlockSpec((1, D), lambda j: (0, 0)),
        ],
        out_specs=pl.BlockSpec((_BKH, 2 * D), lambda j: (j, 0)),
        out_shape=jax.ShapeDtypeStruct((nb * _BKH, 2 * D), jnp.float32),
        compiler_params=pltpu.CompilerParams(
            fuse_transposed_lhs_in_matmul=True,
            vmem_limit_bytes=100 * 1024 * 1024,
        ),
    )(embed.T, W_in, b_in.reshape(1, D))

    return _sc_gather_fold(pt2, idx, D)
